# baseline (speedup 1.0000x reference)
import dataclasses
import functools

import jax
import jax.numpy as jnp
from jax import lax
from jax.experimental import pallas as pl
from jax.experimental.pallas import tpu as pltpu
from jax.experimental.pallas import tpu_sc as plsc

K = 8192
D = 32
N = 32768
BETA = 0.25

HW = 1024
KC = 2048
N_CHUNKS = K // KC
SSTRIPS = KC // 8

SC_CORES = 2
SC_SUBCORES = 16
SC_WORKERS = SC_CORES * SC_SUBCORES
ROWS_PER_WORKER = N // SC_WORKERS
IDX_MINOR = 128
IDX_ROWS_PER_WORKER = ROWS_PER_WORKER // IDX_MINOR


def _argmin_body(lat_ref, embt2_ref, idx_ref, loss_ref):
    f2 = lat_ref[...].reshape(D, HW)
    sumf = jnp.sum(f2 * f2, axis=0, keepdims=True)
    sub8 = lax.broadcasted_iota(jnp.int32, (8, 1), 0).astype(jnp.float32)
    big = jnp.float32(2.0**30)

    candmin = jnp.full((8, HW), jnp.inf, jnp.float32)
    candidx = jnp.zeros((8, HW), jnp.float32)
    for j in range(N_CHUNKS):
        m2t = lax.dot_general(embt2_ref[:, j * KC:(j + 1) * KC], f2,
                              (((0,), (0,)), ((), ())),
                              preferred_element_type=jnp.float32)
        for s in range(SSTRIPS):
            d = sumf - m2t[s * 8:(s + 1) * 8, :]
            msk = d < candmin
            candmin = jnp.where(msk, d, candmin)
            candidx = jnp.where(msk, jnp.float32(j * SSTRIPS + s), candidx)

    gmin = jnp.min(candmin, axis=0, keepdims=True)
    codef = candidx * jnp.float32(8.0) + sub8
    tie = jnp.where(candmin == gmin, codef, big)
    colidx = jnp.min(tie, axis=0, keepdims=True)
    idx_ref[...] = colidx.astype(jnp.int32).reshape(1, 1, HW)

    i = pl.program_id(0)

    @pl.when(i == 0)
    def _init():
        loss_ref[...] = jnp.zeros_like(loss_ref)

    loss_ref[...] += jnp.sum(gmin).reshape(1, 1)


_argmin_call = pl.pallas_call(
    _argmin_body,
    grid=(32,),
    in_specs=[
        pl.BlockSpec((1, D, HW), lambda i: (i, 0, 0)),
        pl.BlockSpec((D, K), lambda i: (0, 0)),
    ],
    out_specs=[
        pl.BlockSpec((1, 1, HW), lambda i: (i, 0, 0)),
        pl.BlockSpec((1, 1), lambda i: (0, 0)),
    ],
    out_shape=[
        jax.ShapeDtypeStruct((32, 1, HW), jnp.int32),
        jax.ShapeDtypeStruct((1, 1), jnp.float32),
    ],
    compiler_params=pltpu.CompilerParams(
        dimension_semantics=("arbitrary",),
    ),
)


def _gather_body(emb_hbm, idx_hbm, out_hbm, idx_v, rows_v, rowst_v, sem):
    c = lax.axis_index("c")
    s = lax.axis_index("s")
    wid = s * SC_CORES + c
    pltpu.sync_copy(idx_hbm.at[pl.ds(wid * IDX_ROWS_PER_WORKER,
                                     IDX_ROWS_PER_WORKER)], idx_v)
    copies = []
    for j in range(IDX_ROWS_PER_WORKER):
        copies.append(pltpu.async_copy(
            emb_hbm.at[idx_v.at[j]],
            rows_v.at[pl.ds(j * IDX_MINOR, IDX_MINOR)],
            sem))
    for cp in copies:
        cp.wait()

    lane = lax.broadcasted_iota(jnp.int32, (16,), 0)
    cidx_lo = lane
    cidx_hi = lane + 16

    def _row8(i, carry):
        r0 = i * 8
        for u in range(8):
            r = r0 + u
            hwidx = jnp.zeros((16,), jnp.int32) + r
            lo = rows_v[r, 0:16]
            hi = rows_v[r, 16:32]
            plsc.store_scatter(rowst_v, [cidx_lo, hwidx], lo)
            plsc.store_scatter(rowst_v, [cidx_hi, hwidx], hi)
        return carry

    lax.fori_loop(0, ROWS_PER_WORKER // 8, _row8, 0)
    pltpu.sync_copy(rowst_v, out_hbm.at[wid])


_gather_call = pl.kernel(
    _gather_body,
    out_type=jax.ShapeDtypeStruct((32, D, HW), jnp.float32),
    mesh=plsc.VectorSubcoreMesh(core_axis_name="c", subcore_axis_name="s"),
    scratch_types=[
        pltpu.VMEM((IDX_ROWS_PER_WORKER, IDX_MINOR), jnp.int32),
        pltpu.VMEM((ROWS_PER_WORKER, D), jnp.float32),
        pltpu.VMEM((D, HW), jnp.float32),
        pltpu.SemaphoreType.DMA,
    ],
    compiler_params=dataclasses.replace(
        pltpu.CompilerParams(use_tc_tiling_on_sc=False),
        **({"needs_layout_passes": False}
           if "needs_layout_passes" in pltpu.CompilerParams.__dataclass_fields__
           else {})),
)


def kernel(latents, embedding):
    lat3 = latents.reshape(32, D, HW)
    embt2 = jnp.transpose(embedding * jnp.float32(2.0))

    idx, loss_sum = _argmin_call(lat3, embt2)

    out3 = _gather_call(embedding, idx.reshape(N // IDX_MINOR, IDX_MINOR))

    out = out3.reshape(32, D, 32, 32)
    vq_loss = loss_sum[0, 0] * jnp.float32((1.0 + BETA) / (N * D))
    return out, vq_loss


# speedup vs baseline: 1.0491x; 1.0491x over previous
import dataclasses
import functools

import jax
import jax.numpy as jnp
from jax import lax
from jax.experimental import pallas as pl
from jax.experimental.pallas import tpu as pltpu
from jax.experimental.pallas import tpu_sc as plsc

K = 8192
D = 32
N = 32768
BETA = 0.25

HW = 1024
KC = 2048
N_CHUNKS = K // KC
SSTRIPS = KC // 8

SC_CORES = 2
SC_SUBCORES = 16
SC_WORKERS = SC_CORES * SC_SUBCORES
ROWS_PER_WORKER = N // SC_WORKERS
IDX_MINOR = 128
IDX_ROWS_PER_WORKER = ROWS_PER_WORKER // IDX_MINOR


def _argmin_body(lat_ref, embt2_ref, idx_ref, loss_ref):
    f2 = lat_ref[...].reshape(D, HW)
    sumf = jnp.sum(f2 * f2, axis=0, keepdims=True)
    sub8 = lax.broadcasted_iota(jnp.int32, (8, 1), 0).astype(jnp.float32)
    big = jnp.float32(2.0**30)

    candmin = jnp.full((8, HW), jnp.inf, jnp.float32)
    candidx = jnp.zeros((8, HW), jnp.float32)
    for j in range(N_CHUNKS):
        m2t = lax.dot_general(embt2_ref[:, j * KC:(j + 1) * KC], f2,
                              (((0,), (0,)), ((), ())),
                              preferred_element_type=jnp.float32)
        for s in range(SSTRIPS):
            d = sumf - m2t[s * 8:(s + 1) * 8, :]
            msk = d < candmin
            candmin = jnp.where(msk, d, candmin)
            candidx = jnp.where(msk, jnp.float32(j * SSTRIPS + s), candidx)

    gmin = jnp.min(candmin, axis=0, keepdims=True)
    codef = candidx * jnp.float32(8.0) + sub8
    tie = jnp.where(candmin == gmin, codef, big)
    colidx = jnp.min(tie, axis=0, keepdims=True)
    idx_ref[...] = colidx.astype(jnp.int32).reshape(1, 1, HW)

    i = pl.program_id(0)

    @pl.when(i == 0)
    def _init():
        loss_ref[...] = jnp.zeros_like(loss_ref)

    loss_ref[...] += jnp.sum(gmin).reshape(1, 1)


_argmin_call = pl.pallas_call(
    _argmin_body,
    grid=(32,),
    in_specs=[
        pl.BlockSpec((1, D, HW), lambda i: (i, 0, 0)),
        pl.BlockSpec((D, K), lambda i: (0, 0)),
    ],
    out_specs=[
        pl.BlockSpec((1, 1, HW), lambda i: (i, 0, 0)),
        pl.BlockSpec((1, 1), lambda i: (0, 0)),
    ],
    out_shape=[
        jax.ShapeDtypeStruct((32, 1, HW), jnp.int32),
        jax.ShapeDtypeStruct((1, 1), jnp.float32),
    ],
    compiler_params=pltpu.CompilerParams(
        dimension_semantics=("arbitrary",),
    ),
)


def _gather_body(emb_hbm, idx_hbm, out_hbm, idx_v, rows_v, rowst_v, sem):
    c = lax.axis_index("c")
    s = lax.axis_index("s")
    wid = s * SC_CORES + c
    pltpu.sync_copy(idx_hbm.at[pl.ds(wid * IDX_ROWS_PER_WORKER,
                                     IDX_ROWS_PER_WORKER)], idx_v)
    copies = []
    for j in range(IDX_ROWS_PER_WORKER):
        copies.append(pltpu.async_copy(
            emb_hbm.at[idx_v.at[j]],
            rows_v.at[pl.ds(j * IDX_MINOR, IDX_MINOR)],
            sem))
    for cp in copies:
        cp.wait()

    lane = lax.broadcasted_iota(jnp.int32, (16,), 0)
    cvecs = [((lane + k) & 15) + c0 for k in range(16) for c0 in (0, 16)]

    def _rowgrp(i, carry):
        rvec = lane + i * 16
        for cvec in cvecs:
            v = plsc.load_gather(rows_v, [rvec, cvec])
            plsc.store_scatter(rowst_v, [cvec, rvec], v)
        return carry

    lax.fori_loop(0, ROWS_PER_WORKER // 16, _rowgrp, 0)
    pltpu.sync_copy(rowst_v, out_hbm.at[wid])


_gather_call = pl.kernel(
    _gather_body,
    out_type=jax.ShapeDtypeStruct((32, D, HW), jnp.float32),
    mesh=plsc.VectorSubcoreMesh(core_axis_name="c", subcore_axis_name="s"),
    scratch_types=[
        pltpu.VMEM((IDX_ROWS_PER_WORKER, IDX_MINOR), jnp.int32),
        pltpu.VMEM((ROWS_PER_WORKER, D), jnp.float32),
        pltpu.VMEM((D, HW), jnp.float32),
        pltpu.SemaphoreType.DMA,
    ],
    compiler_params=dataclasses.replace(
        pltpu.CompilerParams(use_tc_tiling_on_sc=False),
        **({"needs_layout_passes": False}
           if "needs_layout_passes" in pltpu.CompilerParams.__dataclass_fields__
           else {})),
)


def kernel(latents, embedding):
    lat3 = latents.reshape(32, D, HW)
    embt2 = jnp.transpose(embedding * jnp.float32(2.0))

    idx, loss_sum = _argmin_call(lat3, embt2)

    out3 = _gather_call(embedding, idx.reshape(N // IDX_MINOR, IDX_MINOR))

    out = out3.reshape(32, D, 32, 32)
    vq_loss = loss_sum[0, 0] * jnp.float32((1.0 + BETA) / (N * D))
    return out, vq_loss
